# Initial kernel scaffold; baseline (speedup 1.0000x reference)
import jax
import jax.numpy as jnp
from jax.experimental import pallas as pl


def kernel(token_p1, token_p2, token_p3, llm_p1, llm_p2, llm_p3, xyz_p1, xyz_p2, xyz_p3, nuv_p1, nuv_p2, nuv_p3, topk_p1, topk_p2, topk_p3, topk_i2, topk_i3, batch_p1, params):
    raise NotImplementedError("write your pallas kernel here")


# speedup vs baseline: 2.3036x; 2.3036x over previous
import functools

import jax
import jax.numpy as jnp
from jax import lax
from jax.experimental import pallas as pl
from jax.experimental.pallas import tpu as pltpu
from jax.experimental.pallas import tpu_sc as plsc

K = 16
E = 64
H = 4
DH = E // H
DPAD = 128

_HI = jax.lax.Precision.HIGHEST


def _elu(x):
    return jnp.where(x > 0, x, jnp.exp(x) - 1.0)


def _embed_body(tok_ref, llm_ref, emb_ref, g_ref, b_ref, w1_ref, b1_ref,
                w2_ref, b2_ref, out_ref):
    x = llm_ref[...]
    m = jnp.mean(x, axis=1, keepdims=True)
    xc = x - m
    v = jnp.mean(xc * xc, axis=1, keepdims=True)
    h = xc * lax.rsqrt(v + 1e-5) * g_ref[...] + b_ref[...]
    h = _elu(jnp.dot(h, w1_ref[...], preferred_element_type=jnp.float32)
             + b1_ref[...])
    h = _elu(jnp.dot(h, w2_ref[...], preferred_element_type=jnp.float32)
             + b2_ref[...])
    tok = tok_ref[...]
    blk = tok.shape[0]
    oh = (tok == lax.broadcasted_iota(jnp.int32, (blk, 32), 1)
          ).astype(jnp.float32)
    ft = jnp.dot(oh, emb_ref[...], precision=_HI,
                 preferred_element_type=jnp.float32)
    out_ref[...] = jnp.concatenate([ft, h], axis=1)


def _embed(tok2d, llm, emb_pad, ln_g, ln_b, w1, b1, w2, b2):
    n, d = llm.shape
    blk = 1000
    assert n % blk == 0
    return pl.pallas_call(
        _embed_body,
        grid=(n // blk,),
        in_specs=[
            pl.BlockSpec((blk, 1), lambda i: (i, 0)),
            pl.BlockSpec((blk, d), lambda i: (i, 0)),
            pl.BlockSpec((32, 32), lambda i: (0, 0)),
            pl.BlockSpec((1, d), lambda i: (0, 0)),
            pl.BlockSpec((1, d), lambda i: (0, 0)),
            pl.BlockSpec((d, E), lambda i: (0, 0)),
            pl.BlockSpec((1, E), lambda i: (0, 0)),
            pl.BlockSpec((E, 32), lambda i: (0, 0)),
            pl.BlockSpec((1, 32), lambda i: (0, 0)),
        ],
        out_specs=pl.BlockSpec((blk, E), lambda i: (i, 0)),
        out_shape=jax.ShapeDtypeStruct((n, E), jnp.float32),
    )(tok2d, llm, emb_pad, ln_g, ln_b, w1, b1, w2, b2)


def _sc_gather(table, idx):
    bidx = idx.shape[0]
    info = plsc.get_sparse_core_info()
    nw = info.num_cores * info.num_subcores
    per_w = bidx // nw
    assert per_w * nw == bidx
    ch = 760
    while per_w % ch or ch % 8:
        ch -= 8
    nchunk = per_w // ch
    idx3 = idx.reshape(nw, nchunk, ch)
    mesh = plsc.VectorSubcoreMesh(core_axis_name="c", subcore_axis_name="s")

    @functools.partial(
        pl.kernel, mesh=mesh,
        out_type=jax.ShapeDtypeStruct((nw, nchunk, ch, DPAD), jnp.float32),
        scratch_types=[
            pltpu.VMEM((ch,), jnp.int32),
            pltpu.VMEM((ch, DPAD), jnp.float32),
            pltpu.SemaphoreType.DMA,
        ],
    )
    def k(table_hbm, idx_hbm, out_hbm, idx_v, rows_v, sem):
        wid = lax.axis_index("s") * info.num_cores + lax.axis_index("c")

        def body(i, carry):
            pltpu.sync_copy(idx_hbm.at[wid, i], idx_v)
            pltpu.async_copy(table_hbm.at[idx_v], rows_v, sem).wait()
            pltpu.sync_copy(rows_v, out_hbm.at[wid, i])
            return carry

        lax.fori_loop(0, nchunk, body, 0)

    out = k(table, idx3)
    return out.reshape(bidx, DPAD)


def _attn_body(fq_ref, g_ref, xyz_ref, nuv_ref, wq_ref, wk_ref, wv_ref,
               wo_ref, wg_ref, out_ref):
    G = g_ref[...]
    blk = G.shape[0]
    fq = fq_ref[...]
    fg2 = G[:, :, :E].reshape(blk * K, E)
    nk = jnp.dot(fg2, wk_ref[...],
                 preferred_element_type=jnp.float32).reshape(blk, K, E)
    nv = jnp.dot(fg2, wv_ref[...],
                 preferred_element_type=jnp.float32).reshape(blk, K, E)
    xyzg = G[:, :, E:E + 3]
    xyzq = xyz_ref[...]
    nuv = nuv_ref[...]
    rel = xyzg - xyzq[:, None, :]
    wg = wg_ref[...]
    bias = jnp.zeros((blk, K, E), jnp.float32)
    for i in range(3):
        loc = jnp.zeros((blk, K), jnp.float32)
        for j in range(3):
            loc = loc + nuv[:, 3 * i + j][:, None] * rel[:, :, j]
        bias = bias + loc[:, :, None] * wg[i, :][None, None, :]
    nk = nk + bias
    q = jnp.dot(fq, wq_ref[...], preferred_element_type=jnp.float32)
    outs = []
    for h in range(H):
        qh = q[:, DH * h:DH * (h + 1)]
        nkh = nk[:, :, DH * h:DH * (h + 1)]
        nvh = nv[:, :, DH * h:DH * (h + 1)]
        s = jnp.sum(qh[:, None, :] * nkh, axis=2) * 0.25
        s = s - jnp.max(s, axis=1, keepdims=True)
        e = jnp.exp(s)
        a = e / jnp.sum(e, axis=1, keepdims=True)
        outs.append(jnp.sum(a[:, :, None] * nvh, axis=1))
    o = jnp.concatenate(outs, axis=1)
    out_ref[...] = fq + jnp.dot(o, wo_ref[...],
                                preferred_element_type=jnp.float32)


def _attention(fq, g3, xyzq, nuv9, wq, wk, wv, wo, wg):
    m = fq.shape[0]
    blk = 400
    assert m % blk == 0
    return pl.pallas_call(
        _attn_body,
        grid=(m // blk,),
        in_specs=[
            pl.BlockSpec((blk, E), lambda i: (i, 0)),
            pl.BlockSpec((blk, K, DPAD), lambda i: (i, 0, 0)),
            pl.BlockSpec((blk, 3), lambda i: (i, 0)),
            pl.BlockSpec((blk, 9), lambda i: (i, 0)),
            pl.BlockSpec((E, E), lambda i: (0, 0)),
            pl.BlockSpec((E, E), lambda i: (0, 0)),
            pl.BlockSpec((E, E), lambda i: (0, 0)),
            pl.BlockSpec((E, E), lambda i: (0, 0)),
            pl.BlockSpec((3, E), lambda i: (0, 0)),
        ],
        out_specs=pl.BlockSpec((blk, E), lambda i: (i, 0)),
        out_shape=jax.ShapeDtypeStruct((m, E), jnp.float32),
    )(fq, g3, xyzq, nuv9, wq, wk, wv, wo, wg)


def _pool_body(g1_ref, g2_ref, b_ref, w1t_ref, w2t_ref, w3t_ref, out_ref,
               acc_ref):
    i = pl.program_id(0)
    ng = pl.num_programs(0)

    @pl.when(i == 0)
    def _():
        acc_ref[...] = jnp.zeros_like(acc_ref)

    diff = g1_ref[...] - g2_ref[...]
    b = b_ref[...]
    blk = b.shape[0]
    oh = (b == lax.broadcasted_iota(jnp.int32, (blk, 16), 1)
          ).astype(jnp.float32)
    acc_ref[...] += lax.dot_general(diff, oh, (((0,), (0,)), ((), ())),
                                    precision=_HI,
                                    preferred_element_type=jnp.float32)

    @pl.when(i == ng - 1)
    def _():
        t = jnp.dot(w1t_ref[...], acc_ref[...],
                    preferred_element_type=jnp.float32)
        t = jnp.dot(w2t_ref[...], t, preferred_element_type=jnp.float32)
        out_ref[...] = jnp.dot(w3t_ref[...], t,
                               preferred_element_type=jnp.float32)


def _pool_head(g1, g2, batch2d, w1t, w2t, w3t):
    n = g1.shape[0]
    blk = 1000
    assert n % blk == 0
    out = pl.pallas_call(
        _pool_body,
        grid=(n // blk,),
        in_specs=[
            pl.BlockSpec((blk, E), lambda i: (i, 0)),
            pl.BlockSpec((blk, E), lambda i: (i, 0)),
            pl.BlockSpec((blk, 1), lambda i: (i, 0)),
            pl.BlockSpec((E, E), lambda i: (0, 0)),
            pl.BlockSpec((E, E), lambda i: (0, 0)),
            pl.BlockSpec((1, E), lambda i: (0, 0)),
        ],
        out_specs=pl.BlockSpec((1, 16), lambda i: (0, 0)),
        out_shape=jax.ShapeDtypeStruct((1, 16), jnp.float32),
        scratch_shapes=[pltpu.VMEM((E, 16), jnp.float32)],
    )(g1, g2, batch2d, w1t, w2t, w3t)
    return out.reshape(16)


def _make_table(f, xyz):
    n = f.shape[0]
    pad = jnp.zeros((n, DPAD - E - 3), jnp.float32)
    return jnp.concatenate([f, xyz, pad], axis=1)


def kernel(token_p1, token_p2, token_p3, llm_p1, llm_p2, llm_p3, xyz_p1,
           xyz_p2, xyz_p3, nuv_p1, nuv_p2, nuv_p3, topk_p1, topk_p2, topk_p3,
           topk_i2, topk_i3, batch_p1, params):
    p = params
    n = llm_p1.shape[0]
    emb_pad = jnp.pad(p['emb_tok'].astype(jnp.float32), ((0, 11), (0, 0)))
    ln_g = p['ln_g'].reshape(1, -1)
    ln_b = p['ln_b'].reshape(1, -1)
    b1 = p['llm_b1'].reshape(1, -1)
    b2 = p['llm_b2'].reshape(1, -1)

    feats = []
    for tok, llm in ((token_p1, llm_p1), (token_p2, llm_p2),
                     (token_p3, llm_p3)):
        feats.append(_embed(tok.astype(jnp.int32).reshape(n, 1), llm,
                            emb_pad, ln_g, ln_b, p['llm_w1'], b1,
                            p['llm_w2'], b2))
    f1, f2, f3 = feats

    xyzs = (xyz_p1, xyz_p2, xyz_p3)
    nuv_all = jnp.concatenate([nuv_p1.reshape(n, 9), nuv_p2.reshape(n, 9),
                               nuv_p3.reshape(n, 9)], axis=0)
    xyz_all = jnp.concatenate(xyzs, axis=0)
    idx_geo = jnp.concatenate(
        [topk_p1.astype(jnp.int32).reshape(-1),
         topk_p2.astype(jnp.int32).reshape(-1) + n,
         topk_p3.astype(jnp.int32).reshape(-1) + 2 * n], axis=0)

    for l in range(p['stru_Wq'].shape[0]):
        table = jnp.concatenate([_make_table(f1, xyz_p1),
                                 _make_table(f2, xyz_p2),
                                 _make_table(f3, xyz_p3)], axis=0)
        g = _sc_gather(table, idx_geo).reshape(3 * n, K, DPAD)
        fq = jnp.concatenate([f1, f2, f3], axis=0)
        fnew = _attention(fq, g, xyz_all, nuv_all, p['stru_Wq'][l],
                          p['stru_Wk'][l], p['stru_Wv'][l], p['stru_Wo'][l],
                          p['stru_Wg'][l])
        f1, f2, f3 = fnew[:n], fnew[n:2 * n], fnew[2 * n:]

    table_i = jnp.concatenate([_make_table(f2, xyz_p2),
                               _make_table(f3, xyz_p3)], axis=0)
    idx_i = jnp.concatenate([topk_i2.astype(jnp.int32).reshape(-1),
                             topk_i3.astype(jnp.int32).reshape(-1) + n],
                            axis=0)
    gi = _sc_gather(table_i, idx_i).reshape(2 * n, K, DPAD)
    fq_i = jnp.concatenate([f1, f1], axis=0)
    xyz_i = jnp.concatenate([xyz_p1, xyz_p1], axis=0)
    nuv_i = jnp.concatenate([nuv_p1.reshape(n, 9)] * 2, axis=0)
    gout = _attention(fq_i, gi, xyz_i, nuv_i, p['inter_Wq'], p['inter_Wk'],
                      p['inter_Wv'], p['inter_Wo'], p['inter_Wg'])
    g1, g2 = gout[:n], gout[n:]

    return _pool_head(g1, g2, batch_p1.astype(jnp.int32).reshape(n, 1),
                      p['out_w1'].T, p['out_w2'].T, p['out_w3'].T)


# speedup vs baseline: 5.5563x; 2.4120x over previous
import functools

import jax
import jax.numpy as jnp
from jax import lax
from jax.experimental import pallas as pl
from jax.experimental.pallas import tpu as pltpu
from jax.experimental.pallas import tpu_sc as plsc

K = 16
E = 64
H = 4
DH = E // H
DPAD = 128

_HI = jax.lax.Precision.HIGHEST


def _elu(x):
    return jnp.where(x > 0, x, jnp.exp(x) - 1.0)


def _embed_body(tok_ref, llm_ref, xyz_ref, emb_ref, g_ref, b_ref, w1_ref,
                b1_ref, w2_ref, b2_ref, out_ref):
    x = llm_ref[...]
    m = jnp.mean(x, axis=1, keepdims=True)
    xc = x - m
    v = jnp.mean(xc * xc, axis=1, keepdims=True)
    h = xc / jnp.sqrt(v + 1e-5) * g_ref[...] + b_ref[...]
    h = _elu(jnp.dot(h, w1_ref[...], preferred_element_type=jnp.float32)
             + b1_ref[...])
    h = _elu(jnp.dot(h, w2_ref[...], preferred_element_type=jnp.float32)
             + b2_ref[...])
    tok = tok_ref[...]
    blk = tok.shape[0]
    oh = (tok == lax.broadcasted_iota(jnp.int32, (blk, 32), 1)
          ).astype(jnp.float32)
    ft = jnp.dot(oh, emb_ref[...], precision=_HI,
                 preferred_element_type=jnp.float32)
    pad = jnp.zeros((blk, DPAD - E - 3), jnp.float32)
    out_ref[...] = jnp.concatenate([ft, h, xyz_ref[...], pad], axis=1)


def _embed(tok2d, llm, xyz, emb_pad, ln_g, ln_b, w1, b1, w2, b2):
    n, d = llm.shape
    blk = 1000
    assert n % blk == 0
    return pl.pallas_call(
        _embed_body,
        grid=(n // blk,),
        in_specs=[
            pl.BlockSpec((blk, 1), lambda i: (i, 0)),
            pl.BlockSpec((blk, d), lambda i: (i, 0)),
            pl.BlockSpec((blk, 3), lambda i: (i, 0)),
            pl.BlockSpec((32, 32), lambda i: (0, 0)),
            pl.BlockSpec((1, d), lambda i: (0, 0)),
            pl.BlockSpec((1, d), lambda i: (0, 0)),
            pl.BlockSpec((d, E), lambda i: (0, 0)),
            pl.BlockSpec((1, E), lambda i: (0, 0)),
            pl.BlockSpec((E, 32), lambda i: (0, 0)),
            pl.BlockSpec((1, 32), lambda i: (0, 0)),
        ],
        out_specs=pl.BlockSpec((blk, DPAD), lambda i: (i, 0)),
        out_shape=jax.ShapeDtypeStruct((n, DPAD), jnp.float32),
    )(tok2d, llm, xyz, emb_pad, ln_g, ln_b, w1, b1, w2, b2)


def _sc_gather(table, idx):
    bidx = idx.shape[0]
    info = plsc.get_sparse_core_info()
    nw = info.num_cores * info.num_subcores
    per_w = bidx // nw
    assert per_w * nw == bidx
    ch = 760
    while per_w % ch or ch % 8:
        ch -= 8
    nchunk = per_w // ch
    idx3 = idx.reshape(nw, nchunk, ch)
    mesh = plsc.VectorSubcoreMesh(core_axis_name="c", subcore_axis_name="s")

    @functools.partial(
        pl.kernel, mesh=mesh,
        out_type=jax.ShapeDtypeStruct((nw, nchunk, ch, DPAD), jnp.float32),
        scratch_types=[
            pltpu.VMEM((ch,), jnp.int32),
            pltpu.VMEM((ch, DPAD), jnp.float32),
            pltpu.SemaphoreType.DMA,
        ],
    )
    def k(table_hbm, idx_hbm, out_hbm, idx_v, rows_v, sem):
        wid = lax.axis_index("s") * info.num_cores + lax.axis_index("c")

        def body(i, carry):
            pltpu.sync_copy(idx_hbm.at[wid, i], idx_v)
            pltpu.async_copy(table_hbm.at[idx_v], rows_v, sem).wait()
            pltpu.sync_copy(rows_v, out_hbm.at[wid, i])
            return carry

        lax.fori_loop(0, nchunk, body, 0)

    out = k(table, idx3)
    return out.reshape(bidx, DPAD)


def _attn_body(tq_ref, nuv_ref, g_ref, wq_ref, wk_ref, wv_ref, wo_ref,
               wg9_ref, ss_ref, out_ref):
    tq = tq_ref[...]
    blk = tq.shape[0]
    fq = tq[:, :E]
    G = g_ref[...]
    fg = G[:, :E]
    xyzg = G[:, E:E + 3]
    q = jnp.dot(fq, wq_ref[...], preferred_element_type=jnp.float32)
    qx = jnp.concatenate([q, tq[:, E:E + 3], nuv_ref[...]], axis=1)
    qxe = jnp.broadcast_to(qx[:, None, :], (blk, K, E + 12)
                           ).reshape(blk * K, E + 12)
    rel = xyzg - qxe[:, E:E + 3]
    rel3 = jnp.concatenate([rel, rel, rel], axis=1)
    prod = qxe[:, E + 3:E + 12] * rel3
    bias = jnp.dot(prod, wg9_ref[...], preferred_element_type=jnp.float32)
    nk = jnp.dot(fg, wk_ref[...], preferred_element_type=jnp.float32) + bias
    nv = jnp.dot(fg, wv_ref[...], preferred_element_type=jnp.float32)
    p = qxe[:, :E] * nk
    sr = jnp.dot(p, ss_ref[...], preferred_element_type=jnp.float32)
    e3 = jnp.exp(sr).reshape(blk, K, E)
    a3 = e3 / jnp.sum(e3, axis=1, keepdims=True)
    w3 = a3 * nv.reshape(blk, K, E)
    o = jnp.sum(w3, axis=1)
    nf = fq + jnp.dot(o, wo_ref[...], preferred_element_type=jnp.float32)
    out_ref[...] = jnp.concatenate([nf, tq[:, E:]], axis=1)


def _attention(table, nuv9, g2, wq, wk, wv, wo, wg9, ss, m, tq_map,
               blk=400):
    nblk = m // blk
    assert nblk * blk == m
    return pl.pallas_call(
        _attn_body,
        grid=(nblk,),
        in_specs=[
            pl.BlockSpec((blk, DPAD), tq_map),
            pl.BlockSpec((blk, 9), tq_map),
            pl.BlockSpec((blk * K, DPAD), lambda i: (i, 0)),
            pl.BlockSpec((E, E), lambda i: (0, 0)),
            pl.BlockSpec((E, E), lambda i: (0, 0)),
            pl.BlockSpec((E, E), lambda i: (0, 0)),
            pl.BlockSpec((E, E), lambda i: (0, 0)),
            pl.BlockSpec((9, E), lambda i: (0, 0)),
            pl.BlockSpec((E, E), lambda i: (0, 0)),
        ],
        out_specs=pl.BlockSpec((blk, DPAD), lambda i: (i, 0)),
        out_shape=jax.ShapeDtypeStruct((m, DPAD), jnp.float32),
    )(table, nuv9, g2, wq, wk, wv, wo, wg9, ss)


def _pool_body(g1_ref, g2_ref, b_ref, w1t_ref, w2t_ref, w3t_ref, out_ref,
               acc_ref):
    i = pl.program_id(0)
    ng = pl.num_programs(0)

    @pl.when(i == 0)
    def _():
        acc_ref[...] = jnp.zeros_like(acc_ref)

    diff = g1_ref[:, :E] - g2_ref[:, :E]
    b = b_ref[...]
    blk = b.shape[0]
    oh = (b == lax.broadcasted_iota(jnp.int32, (blk, 16), 1)
          ).astype(jnp.float32)
    acc_ref[...] += lax.dot_general(diff, oh, (((0,), (0,)), ((), ())),
                                    precision=_HI,
                                    preferred_element_type=jnp.float32)

    @pl.when(i == ng - 1)
    def _():
        t = jnp.dot(w1t_ref[...], acc_ref[...],
                    preferred_element_type=jnp.float32)
        t = jnp.dot(w2t_ref[...], t, preferred_element_type=jnp.float32)
        out_ref[...] = jnp.dot(w3t_ref[...], t,
                               preferred_element_type=jnp.float32)


def _pool_head(gout, batch2d, w1t, w2t, w3t):
    n = batch2d.shape[0]
    blk = 1000
    assert n % blk == 0
    nblk = n // blk
    out = pl.pallas_call(
        _pool_body,
        grid=(nblk,),
        in_specs=[
            pl.BlockSpec((blk, DPAD), lambda i: (i, 0)),
            pl.BlockSpec((blk, DPAD), lambda i: (i + nblk, 0)),
            pl.BlockSpec((blk, 1), lambda i: (i, 0)),
            pl.BlockSpec((E, E), lambda i: (0, 0)),
            pl.BlockSpec((E, E), lambda i: (0, 0)),
            pl.BlockSpec((1, E), lambda i: (0, 0)),
        ],
        out_specs=pl.BlockSpec((1, 16), lambda i: (0, 0)),
        out_shape=jax.ShapeDtypeStruct((1, 16), jnp.float32),
        scratch_shapes=[pltpu.VMEM((E, 16), jnp.float32)],
    )(gout, gout, batch2d, w1t, w2t, w3t)
    return out.reshape(16)


def kernel(token_p1, token_p2, token_p3, llm_p1, llm_p2, llm_p3, xyz_p1,
           xyz_p2, xyz_p3, nuv_p1, nuv_p2, nuv_p3, topk_p1, topk_p2, topk_p3,
           topk_i2, topk_i3, batch_p1, params):
    p = params
    n = llm_p1.shape[0]
    emb_pad = jnp.pad(p['emb_tok'].astype(jnp.float32), ((0, 11), (0, 0)))
    ln_g = p['ln_g'].reshape(1, -1)
    ln_b = p['ln_b'].reshape(1, -1)
    b1 = p['llm_b1'].reshape(1, -1)
    b2 = p['llm_b2'].reshape(1, -1)
    ss = jnp.kron(jnp.eye(H, dtype=jnp.float32),
                  jnp.ones((DH, DH), jnp.float32))
    scale = 1.0 / (DH ** 0.5)

    tabs = []
    for tok, llm, xyz in ((token_p1, llm_p1, xyz_p1),
                          (token_p2, llm_p2, xyz_p2),
                          (token_p3, llm_p3, xyz_p3)):
        tabs.append(_embed(tok.astype(jnp.int32).reshape(n, 1), llm, xyz,
                           emb_pad, ln_g, ln_b, p['llm_w1'], b1,
                           p['llm_w2'], b2))
    table = jnp.concatenate(tabs, axis=0)

    nuv_all = jnp.concatenate([nuv_p1.reshape(n, 9), nuv_p2.reshape(n, 9),
                               nuv_p3.reshape(n, 9)], axis=0)
    idx_geo = jnp.concatenate(
        [topk_p1.astype(jnp.int32).reshape(-1),
         topk_p2.astype(jnp.int32).reshape(-1) + n,
         topk_p3.astype(jnp.int32).reshape(-1) + 2 * n], axis=0)

    for l in range(p['stru_Wq'].shape[0]):
        g2 = _sc_gather(table, idx_geo)
        table = _attention(table, nuv_all, g2, p['stru_Wq'][l] * scale,
                           p['stru_Wk'][l], p['stru_Wv'][l],
                           p['stru_Wo'][l],
                           jnp.repeat(p['stru_Wg'][l], 3, axis=0),
                           ss, 3 * n, lambda i: (i, 0))

    idx_i = jnp.concatenate(
        [topk_i2.astype(jnp.int32).reshape(-1) + n,
         topk_i3.astype(jnp.int32).reshape(-1) + 2 * n], axis=0)
    gi = _sc_gather(table, idx_i)
    nq = (n // 400)
    gout = _attention(table, nuv_all, gi, p['inter_Wq'] * scale,
                      p['inter_Wk'], p['inter_Wv'], p['inter_Wo'],
                      jnp.repeat(p['inter_Wg'], 3, axis=0),
                      ss, 2 * n, lambda i: (i % nq, 0))

    return _pool_head(gout, batch_p1.astype(jnp.int32).reshape(n, 1),
                      p['out_w1'].T, p['out_w2'].T, p['out_w3'].T)


# speedup vs baseline: 9.2159x; 1.6587x over previous
import functools

import jax
import jax.numpy as jnp
from jax import lax
from jax.experimental import pallas as pl
from jax.experimental.pallas import tpu as pltpu
from jax.experimental.pallas import tpu_sc as plsc

K = 16
E = 64
H = 4
DH = E // H
DPAD = 128

_HI = jax.lax.Precision.HIGHEST


def _elu(x):
    return jnp.where(x > 0, x, jnp.exp(x) - 1.0)


def _embed_body(tok_ref, llm_ref, xyz_ref, emb_ref, g_ref, b_ref, w1_ref,
                b1_ref, w2_ref, b2_ref, out_ref):
    x = llm_ref[...]
    m = jnp.mean(x, axis=1, keepdims=True)
    xc = x - m
    v = jnp.mean(xc * xc, axis=1, keepdims=True)
    h = xc / jnp.sqrt(v + 1e-5) * g_ref[...] + b_ref[...]
    h = _elu(jnp.dot(h, w1_ref[...], preferred_element_type=jnp.float32)
             + b1_ref[...])
    h = _elu(jnp.dot(h, w2_ref[...], preferred_element_type=jnp.float32)
             + b2_ref[...])
    tok = tok_ref[...]
    blk = tok.shape[0]
    oh = (tok == lax.broadcasted_iota(jnp.int32, (blk, 32), 1)
          ).astype(jnp.float32)
    ft = jnp.dot(oh, emb_ref[...], precision=_HI,
                 preferred_element_type=jnp.float32)
    xyz = xyz_ref[...]
    pad = jnp.zeros((blk, DPAD - E - 9), jnp.float32)
    out_ref[...] = jnp.concatenate([ft, h, xyz, xyz, xyz, pad], axis=1)


def _embed(tok2d, llm, xyz, emb_pad, ln_g, ln_b, w1, b1, w2, b2):
    n, d = llm.shape
    blk = 1000
    assert n % blk == 0
    return pl.pallas_call(
        _embed_body,
        grid=(n // blk,),
        in_specs=[
            pl.BlockSpec((blk, 1), lambda i: (i, 0)),
            pl.BlockSpec((blk, d), lambda i: (i, 0)),
            pl.BlockSpec((blk, 3), lambda i: (i, 0)),
            pl.BlockSpec((32, 32), lambda i: (0, 0)),
            pl.BlockSpec((1, d), lambda i: (0, 0)),
            pl.BlockSpec((1, d), lambda i: (0, 0)),
            pl.BlockSpec((d, E), lambda i: (0, 0)),
            pl.BlockSpec((1, E), lambda i: (0, 0)),
            pl.BlockSpec((E, 32), lambda i: (0, 0)),
            pl.BlockSpec((1, 32), lambda i: (0, 0)),
        ],
        out_specs=pl.BlockSpec((blk, DPAD), lambda i: (i, 0)),
        out_shape=jax.ShapeDtypeStruct((n, DPAD), jnp.float32),
    )(tok2d, llm, xyz, emb_pad, ln_g, ln_b, w1, b1, w2, b2)


def _sc_gather(table, idx):
    bidx = idx.shape[0]
    info = plsc.get_sparse_core_info()
    nw = info.num_cores * info.num_subcores
    per_w = bidx // nw
    assert per_w * nw == bidx
    ch = 440
    while per_w % ch or ch % 8:
        ch -= 8
    nchunk = per_w // ch
    idx3 = idx.reshape(nw, nchunk, ch)
    mesh = plsc.VectorSubcoreMesh(core_axis_name="c", subcore_axis_name="s")

    @functools.partial(
        pl.kernel, mesh=mesh,
        out_type=jax.ShapeDtypeStruct((nw, nchunk, ch, DPAD), jnp.float32),
        scratch_types=[
            pltpu.VMEM((ch,), jnp.int32),
            pltpu.VMEM((ch,), jnp.int32),
            pltpu.VMEM((ch, DPAD), jnp.float32),
            pltpu.VMEM((ch, DPAD), jnp.float32),
            pltpu.SemaphoreType.DMA,
            pltpu.SemaphoreType.DMA,
        ],
    )
    def k(table_hbm, idx_hbm, out_hbm, idx_a, idx_b, buf_a, buf_b, sem_a,
          sem_b):
        wid = lax.axis_index("s") * info.num_cores + lax.axis_index("c")

        pltpu.sync_copy(idx_hbm.at[wid, 0], idx_a)
        pltpu.async_copy(table_hbm.at[idx_a], buf_a, sem_a)

        def pair(h, carry):
            c0 = 2 * h
            pltpu.sync_copy(idx_hbm.at[wid, c0 + 1], idx_b)
            pltpu.async_copy(table_hbm.at[idx_b], buf_b, sem_b)
            pltpu.make_async_copy(table_hbm.at[idx_a], buf_a, sem_a).wait()
            pltpu.sync_copy(buf_a, out_hbm.at[wid, c0])

            @pl.when(c0 + 2 < nchunk)
            def _():
                pltpu.sync_copy(idx_hbm.at[wid, c0 + 2], idx_a)
                pltpu.async_copy(table_hbm.at[idx_a], buf_a, sem_a)

            pltpu.make_async_copy(table_hbm.at[idx_b], buf_b, sem_b).wait()
            pltpu.sync_copy(buf_b, out_hbm.at[wid, c0 + 1])
            return carry

        lax.fori_loop(0, nchunk // 2, pair, 0)
        if nchunk % 2:
            pltpu.make_async_copy(table_hbm.at[idx_a], buf_a, sem_a).wait()
            pltpu.sync_copy(buf_a, out_hbm.at[wid, nchunk - 1])

    out = k(table, idx3)
    return out.reshape(bidx, DPAD)


def _attn_body(tq_ref, nuv_ref, g_ref, wq_ref, wkg_ref, wv2_ref, wo_ref,
               ss_ref, out_ref):
    tq = tq_ref[...]
    blk = tq.shape[0]
    fq = tq[:, :E]
    G = g_ref[...]
    q = jnp.dot(fq, wq_ref[...], preferred_element_type=jnp.float32)
    zf = jnp.zeros((blk, E), jnp.float32)
    pb = jnp.zeros((blk, DPAD - E - 9), jnp.float32)
    qxa = jnp.concatenate([zf, tq[:, E:]], axis=1)
    qxb = jnp.concatenate([zf + 1.0, nuv_ref[...], pb], axis=1)
    qxae = jnp.broadcast_to(qxa[:, None, :], (blk, K, DPAD)
                            ).reshape(blk * K, DPAD)
    qxbe = jnp.broadcast_to(qxb[:, None, :], (blk, K, DPAD)
                            ).reshape(blk * K, DPAD)
    qe = jnp.broadcast_to(q[:, None, :], (blk, K, E)).reshape(blk * K, E)
    x = (G - qxae) * qxbe
    nk = jnp.dot(x, wkg_ref[...], preferred_element_type=jnp.float32)
    nv = jnp.dot(G, wv2_ref[...], preferred_element_type=jnp.float32)
    p = qe * nk
    sr = jnp.dot(p, ss_ref[...], preferred_element_type=jnp.float32)
    e3 = jnp.exp(sr).reshape(blk, K, E)
    r = 1.0 / jnp.sum(e3, axis=1, keepdims=True)
    w3 = (e3 * r) * nv.reshape(blk, K, E)
    o = jnp.sum(w3, axis=1)
    nf = fq + jnp.dot(o, wo_ref[...], preferred_element_type=jnp.float32)
    out_ref[...] = jnp.concatenate([nf, tq[:, E:]], axis=1)


def _attention(table, nuv9, g2, wq, wkg, wv2, wo, ss, m, tq_map,
               blk=400):
    nblk = m // blk
    assert nblk * blk == m
    return pl.pallas_call(
        _attn_body,
        grid=(nblk,),
        in_specs=[
            pl.BlockSpec((blk, DPAD), tq_map),
            pl.BlockSpec((blk, 9), tq_map),
            pl.BlockSpec((blk * K, DPAD), lambda i: (i, 0)),
            pl.BlockSpec((E, E), lambda i: (0, 0)),
            pl.BlockSpec((DPAD, E), lambda i: (0, 0)),
            pl.BlockSpec((DPAD, E), lambda i: (0, 0)),
            pl.BlockSpec((E, E), lambda i: (0, 0)),
            pl.BlockSpec((E, E), lambda i: (0, 0)),
        ],
        out_specs=pl.BlockSpec((blk, DPAD), lambda i: (i, 0)),
        out_shape=jax.ShapeDtypeStruct((m, DPAD), jnp.float32),
    )(table, nuv9, g2, wq, wkg, wv2, wo, ss)


def _pool_body(g1_ref, g2_ref, b_ref, w1t_ref, w2t_ref, w3t_ref, out_ref,
               acc_ref):
    i = pl.program_id(0)
    ng = pl.num_programs(0)

    @pl.when(i == 0)
    def _():
        acc_ref[...] = jnp.zeros_like(acc_ref)

    diff = g1_ref[:, :E] - g2_ref[:, :E]
    b = b_ref[...]
    blk = b.shape[0]
    oh = (b == lax.broadcasted_iota(jnp.int32, (blk, 16), 1)
          ).astype(jnp.float32)
    acc_ref[...] += lax.dot_general(diff, oh, (((0,), (0,)), ((), ())),
                                    precision=_HI,
                                    preferred_element_type=jnp.float32)

    @pl.when(i == ng - 1)
    def _():
        t = jnp.dot(w1t_ref[...], acc_ref[...],
                    preferred_element_type=jnp.float32)
        t = jnp.dot(w2t_ref[...], t, preferred_element_type=jnp.float32)
        out_ref[...] = jnp.dot(w3t_ref[...], t,
                               preferred_element_type=jnp.float32)


def _pool_head(gout, batch2d, w1t, w2t, w3t):
    n = batch2d.shape[0]
    blk = 1000
    assert n % blk == 0
    nblk = n // blk
    out = pl.pallas_call(
        _pool_body,
        grid=(nblk,),
        in_specs=[
            pl.BlockSpec((blk, DPAD), lambda i: (i, 0)),
            pl.BlockSpec((blk, DPAD), lambda i: (i + nblk, 0)),
            pl.BlockSpec((blk, 1), lambda i: (i, 0)),
            pl.BlockSpec((E, E), lambda i: (0, 0)),
            pl.BlockSpec((E, E), lambda i: (0, 0)),
            pl.BlockSpec((1, E), lambda i: (0, 0)),
        ],
        out_specs=pl.BlockSpec((1, 16), lambda i: (0, 0)),
        out_shape=jax.ShapeDtypeStruct((1, 16), jnp.float32),
        scratch_shapes=[pltpu.VMEM((E, 16), jnp.float32)],
    )(gout, gout, batch2d, w1t, w2t, w3t)
    return out.reshape(16)


def kernel(token_p1, token_p2, token_p3, llm_p1, llm_p2, llm_p3, xyz_p1,
           xyz_p2, xyz_p3, nuv_p1, nuv_p2, nuv_p3, topk_p1, topk_p2, topk_p3,
           topk_i2, topk_i3, batch_p1, params):
    p = params
    n = llm_p1.shape[0]
    emb_pad = jnp.pad(p['emb_tok'].astype(jnp.float32), ((0, 11), (0, 0)))
    ln_g = p['ln_g'].reshape(1, -1)
    ln_b = p['ln_b'].reshape(1, -1)
    b1 = p['llm_b1'].reshape(1, -1)
    b2 = p['llm_b2'].reshape(1, -1)
    ss = jnp.kron(jnp.eye(H, dtype=jnp.float32),
                  jnp.ones((DH, DH), jnp.float32))
    scale = 1.0 / (DH ** 0.5)

    tabs = []
    for tok, llm, xyz in ((token_p1, llm_p1, xyz_p1),
                          (token_p2, llm_p2, xyz_p2),
                          (token_p3, llm_p3, xyz_p3)):
        tabs.append(_embed(tok.astype(jnp.int32).reshape(n, 1), llm, xyz,
                           emb_pad, ln_g, ln_b, p['llm_w1'], b1,
                           p['llm_w2'], b2))
    table = jnp.concatenate(tabs, axis=0)

    nuv_all = jnp.concatenate([nuv_p1.reshape(n, 9), nuv_p2.reshape(n, 9),
                               nuv_p3.reshape(n, 9)], axis=0)
    idx_geo = jnp.concatenate(
        [topk_p1.astype(jnp.int32).reshape(-1),
         topk_p2.astype(jnp.int32).reshape(-1) + n,
         topk_p3.astype(jnp.int32).reshape(-1) + 2 * n], axis=0)

    zkg = jnp.zeros((DPAD - E - 9, E), jnp.float32)
    zv = jnp.zeros((DPAD - E, E), jnp.float32)

    def wkg_of(wk, wg):
        return jnp.concatenate([wk, jnp.repeat(wg, 3, axis=0), zkg], axis=0)

    for l in range(p['stru_Wq'].shape[0]):
        g2 = _sc_gather(table, idx_geo)
        table = _attention(table, nuv_all, g2, p['stru_Wq'][l] * scale,
                           wkg_of(p['stru_Wk'][l], p['stru_Wg'][l]),
                           jnp.concatenate([p['stru_Wv'][l], zv], axis=0),
                           p['stru_Wo'][l],
                           ss, 3 * n, lambda i: (i, 0))

    idx_i = jnp.concatenate(
        [topk_i2.astype(jnp.int32).reshape(-1) + n,
         topk_i3.astype(jnp.int32).reshape(-1) + 2 * n], axis=0)
    gi = _sc_gather(table, idx_i)
    nq = (n // 400)
    gout = _attention(table, nuv_all, gi, p['inter_Wq'] * scale,
                      wkg_of(p['inter_Wk'], p['inter_Wg']),
                      jnp.concatenate([p['inter_Wv'], zv], axis=0),
                      p['inter_Wo'], ss, 2 * n, lambda i: (i % nq, 0))

    return _pool_head(gout, batch_p1.astype(jnp.int32).reshape(n, 1),
                      p['out_w1'].T, p['out_w2'].T, p['out_w3'].T)


# speedup vs baseline: 11.1324x; 1.2079x over previous
import functools

import jax
import jax.numpy as jnp
from jax import lax
from jax.experimental import pallas as pl
from jax.experimental.pallas import tpu as pltpu
from jax.experimental.pallas import tpu_sc as plsc

K = 16
E = 64
H = 4
DH = E // H
DPAD = 128

_HI = jax.lax.Precision.HIGHEST


def _elu(x):
    return jnp.where(x > 0, x, jnp.exp(x) - 1.0)


def _embed_body(tok_ref, llm_ref, xyz_ref, emb_ref, g_ref, b_ref, w1_ref,
                b1_ref, w2_ref, b2_ref, out_ref):
    x = llm_ref[...]
    m = jnp.mean(x, axis=1, keepdims=True)
    xc = x - m
    v = jnp.mean(xc * xc, axis=1, keepdims=True)
    h = xc / jnp.sqrt(v + 1e-5) * g_ref[...] + b_ref[...]
    h = _elu(jnp.dot(h, w1_ref[...], preferred_element_type=jnp.float32)
             + b1_ref[...])
    h = _elu(jnp.dot(h, w2_ref[...], preferred_element_type=jnp.float32)
             + b2_ref[...])
    tok = tok_ref[...]
    blk = tok.shape[0]
    oh = (tok == lax.broadcasted_iota(jnp.int32, (blk, 32), 1)
          ).astype(jnp.float32)
    ft = jnp.dot(oh, emb_ref[...], precision=_HI,
                 preferred_element_type=jnp.float32)
    xyz = xyz_ref[...]
    pad = jnp.zeros((blk, DPAD - E - 9), jnp.float32)
    out_ref[...] = jnp.concatenate([ft, h, xyz, xyz, xyz, pad], axis=1)


def _embed(tok2d, llm, xyz, emb_pad, ln_g, ln_b, w1, b1, w2, b2):
    n, d = llm.shape
    blk = 1000
    assert n % blk == 0
    return pl.pallas_call(
        _embed_body,
        grid=(n // blk,),
        in_specs=[
            pl.BlockSpec((blk, 1), lambda i: (i, 0)),
            pl.BlockSpec((blk, d), lambda i: (i, 0)),
            pl.BlockSpec((blk, 3), lambda i: (i, 0)),
            pl.BlockSpec((32, 32), lambda i: (0, 0)),
            pl.BlockSpec((1, d), lambda i: (0, 0)),
            pl.BlockSpec((1, d), lambda i: (0, 0)),
            pl.BlockSpec((d, E), lambda i: (0, 0)),
            pl.BlockSpec((1, E), lambda i: (0, 0)),
            pl.BlockSpec((E, 32), lambda i: (0, 0)),
            pl.BlockSpec((1, 32), lambda i: (0, 0)),
        ],
        out_specs=pl.BlockSpec((blk, DPAD), lambda i: (i, 0)),
        out_shape=jax.ShapeDtypeStruct((n, DPAD), jnp.float32),
    )(tok2d, llm, xyz, emb_pad, ln_g, ln_b, w1, b1, w2, b2)


def _sc_gather(table, idx):
    bidx = idx.shape[0]
    info = plsc.get_sparse_core_info()
    nw = info.num_cores * info.num_subcores
    per_w = bidx // nw
    assert per_w * nw == bidx
    ch = 440
    while per_w % ch or ch % 8:
        ch -= 8
    nchunk = per_w // ch
    idx3 = idx.reshape(nw, nchunk, ch)
    mesh = plsc.VectorSubcoreMesh(core_axis_name="c", subcore_axis_name="s")

    @functools.partial(
        pl.kernel, mesh=mesh,
        out_type=jax.ShapeDtypeStruct((nw, nchunk, ch, DPAD), jnp.float32),
        scratch_types=[
            pltpu.VMEM((ch,), jnp.int32),
            pltpu.VMEM((ch,), jnp.int32),
            pltpu.VMEM((ch, DPAD), jnp.float32),
            pltpu.VMEM((ch, DPAD), jnp.float32),
            pltpu.SemaphoreType.DMA,
            pltpu.SemaphoreType.DMA,
        ],
    )
    def k(table_hbm, idx_hbm, out_hbm, idx_a, idx_b, buf_a, buf_b, sem_a,
          sem_b):
        wid = lax.axis_index("s") * info.num_cores + lax.axis_index("c")

        pltpu.sync_copy(idx_hbm.at[wid, 0], idx_a)
        pltpu.async_copy(table_hbm.at[idx_a], buf_a, sem_a)

        def pair(h, carry):
            c0 = 2 * h
            pltpu.sync_copy(idx_hbm.at[wid, c0 + 1], idx_b)
            pltpu.async_copy(table_hbm.at[idx_b], buf_b, sem_b)
            pltpu.make_async_copy(table_hbm.at[idx_a], buf_a, sem_a).wait()
            pltpu.sync_copy(buf_a, out_hbm.at[wid, c0])

            @pl.when(c0 + 2 < nchunk)
            def _():
                pltpu.sync_copy(idx_hbm.at[wid, c0 + 2], idx_a)
                pltpu.async_copy(table_hbm.at[idx_a], buf_a, sem_a)

            pltpu.make_async_copy(table_hbm.at[idx_b], buf_b, sem_b).wait()
            pltpu.sync_copy(buf_b, out_hbm.at[wid, c0 + 1])
            return carry

        lax.fori_loop(0, nchunk // 2, pair, 0)
        if nchunk % 2:
            pltpu.make_async_copy(table_hbm.at[idx_a], buf_a, sem_a).wait()
            pltpu.sync_copy(buf_a, out_hbm.at[wid, nchunk - 1])

    out = k(table, idx3)
    return out.reshape(bidx, DPAD)


def _attn_body(tq_ref, nuv_ref, g_ref, wq_ref, wkg_ref, wv2_ref, wo_ref,
               ss_ref, out_ref):
    tq = tq_ref[...]
    blk = tq.shape[0]
    fq = tq[:, :E]
    G = g_ref[...]
    q = jnp.dot(fq, wq_ref[...], preferred_element_type=jnp.float32)
    zf = jnp.zeros((blk, E), jnp.float32)
    pb = jnp.zeros((blk, DPAD - E - 9), jnp.float32)
    qxa = jnp.concatenate([zf, tq[:, E:]], axis=1)
    qxb = jnp.concatenate([zf + 1.0, nuv_ref[...], pb], axis=1)
    qxae = jnp.broadcast_to(qxa[:, None, :], (blk, K, DPAD)
                            ).reshape(blk * K, DPAD)
    qxbe = jnp.broadcast_to(qxb[:, None, :], (blk, K, DPAD)
                            ).reshape(blk * K, DPAD)
    qe = jnp.broadcast_to(q[:, None, :], (blk, K, E)).reshape(blk * K, E)
    x = (G - qxae) * qxbe
    nk = jnp.dot(x, wkg_ref[...], preferred_element_type=jnp.float32)
    nv = jnp.dot(G, wv2_ref[...], preferred_element_type=jnp.float32)
    p = qe * nk
    sr = jnp.dot(p, ss_ref[...], preferred_element_type=jnp.float32)
    e3 = jnp.exp(sr).reshape(blk, K, E)
    r = 1.0 / jnp.sum(e3, axis=1, keepdims=True)
    w3 = (e3 * r) * nv.reshape(blk, K, E)
    o = jnp.sum(w3, axis=1)
    nf = fq + jnp.dot(o, wo_ref[...], preferred_element_type=jnp.float32)
    out_ref[...] = jnp.concatenate([nf, tq[:, E:]], axis=1)


def _attention(table, nuv9, g2, wq, wkg, wv2, wo, ss, m, tq_map,
               blk=400):
    nblk = m // blk
    assert nblk * blk == m
    return pl.pallas_call(
        _attn_body,
        grid=(nblk,),
        in_specs=[
            pl.BlockSpec((blk, DPAD), tq_map),
            pl.BlockSpec((blk, 9), tq_map),
            pl.BlockSpec((blk * K, DPAD), lambda i: (i, 0)),
            pl.BlockSpec((E, E), lambda i: (0, 0)),
            pl.BlockSpec((DPAD, E), lambda i: (0, 0)),
            pl.BlockSpec((DPAD, E), lambda i: (0, 0)),
            pl.BlockSpec((E, E), lambda i: (0, 0)),
            pl.BlockSpec((E, E), lambda i: (0, 0)),
        ],
        out_specs=pl.BlockSpec((blk, DPAD), lambda i: (i, 0)),
        out_shape=jax.ShapeDtypeStruct((m, DPAD), jnp.float32),
    )(table, nuv9, g2, wq, wkg, wv2, wo, ss)


def _pool_body(g1_ref, g2_ref, b_ref, w1t_ref, w2t_ref, w3t_ref, out_ref,
               acc_ref):
    i = pl.program_id(0)
    ng = pl.num_programs(0)

    @pl.when(i == 0)
    def _():
        acc_ref[...] = jnp.zeros_like(acc_ref)

    diff = g1_ref[:, :E] - g2_ref[:, :E]
    b = b_ref[...]
    blk = b.shape[0]
    oh = (b == lax.broadcasted_iota(jnp.int32, (blk, 16), 1)
          ).astype(jnp.float32)
    acc_ref[...] += lax.dot_general(diff, oh, (((0,), (0,)), ((), ())),
                                    precision=_HI,
                                    preferred_element_type=jnp.float32)

    @pl.when(i == ng - 1)
    def _():
        t = jnp.dot(w1t_ref[...], acc_ref[...],
                    preferred_element_type=jnp.float32)
        t = jnp.dot(w2t_ref[...], t, preferred_element_type=jnp.float32)
        out_ref[...] = jnp.dot(w3t_ref[...], t,
                               preferred_element_type=jnp.float32)


def _pool_head(g1, g2, batch2d, w1t, w2t, w3t):
    n = batch2d.shape[0]
    blk = 1000
    assert n % blk == 0
    nblk = n // blk
    out = pl.pallas_call(
        _pool_body,
        grid=(nblk,),
        in_specs=[
            pl.BlockSpec((blk, DPAD), lambda i: (i, 0)),
            pl.BlockSpec((blk, DPAD), lambda i: (i, 0)),
            pl.BlockSpec((blk, 1), lambda i: (i, 0)),
            pl.BlockSpec((E, E), lambda i: (0, 0)),
            pl.BlockSpec((E, E), lambda i: (0, 0)),
            pl.BlockSpec((1, E), lambda i: (0, 0)),
        ],
        out_specs=pl.BlockSpec((1, 16), lambda i: (0, 0)),
        out_shape=jax.ShapeDtypeStruct((1, 16), jnp.float32),
        scratch_shapes=[pltpu.VMEM((E, 16), jnp.float32)],
    )(g1, g2, batch2d, w1t, w2t, w3t)
    return out.reshape(16)


def kernel(token_p1, token_p2, token_p3, llm_p1, llm_p2, llm_p3, xyz_p1,
           xyz_p2, xyz_p3, nuv_p1, nuv_p2, nuv_p3, topk_p1, topk_p2, topk_p3,
           topk_i2, topk_i3, batch_p1, params):
    p = params
    n = llm_p1.shape[0]
    emb_pad = jnp.pad(p['emb_tok'].astype(jnp.float32), ((0, 11), (0, 0)))
    ln_g = p['ln_g'].reshape(1, -1)
    ln_b = p['ln_b'].reshape(1, -1)
    b1 = p['llm_b1'].reshape(1, -1)
    b2 = p['llm_b2'].reshape(1, -1)
    ss = jnp.kron(jnp.eye(H, dtype=jnp.float32),
                  jnp.ones((DH, DH), jnp.float32))
    scale = 1.0 / (DH ** 0.5)

    tabs = []
    for tok, llm, xyz in ((token_p1, llm_p1, xyz_p1),
                          (token_p2, llm_p2, xyz_p2),
                          (token_p3, llm_p3, xyz_p3)):
        tabs.append(_embed(tok.astype(jnp.int32).reshape(n, 1), llm, xyz,
                           emb_pad, ln_g, ln_b, p['llm_w1'], b1,
                           p['llm_w2'], b2))

    nuvs = [nuv_p1.reshape(n, 9), nuv_p2.reshape(n, 9), nuv_p3.reshape(n, 9)]
    idxs = [topk_p1.astype(jnp.int32).reshape(-1),
            topk_p2.astype(jnp.int32).reshape(-1),
            topk_p3.astype(jnp.int32).reshape(-1)]

    zkg = jnp.zeros((DPAD - E - 9, E), jnp.float32)
    zv = jnp.zeros((DPAD - E, E), jnp.float32)

    def wkg_of(wk, wg):
        return jnp.concatenate([wk, jnp.repeat(wg, 3, axis=0), zkg], axis=0)

    ident = lambda i: (i, 0)
    for l in range(p['stru_Wq'].shape[0]):
        wq = p['stru_Wq'][l] * scale
        wkg = wkg_of(p['stru_Wk'][l], p['stru_Wg'][l])
        wv2 = jnp.concatenate([p['stru_Wv'][l], zv], axis=0)
        wo = p['stru_Wo'][l]
        gs = [_sc_gather(tabs[c], idxs[c]) for c in range(3)]
        tabs = [_attention(tabs[c], nuvs[c], gs[c], wq, wkg, wv2, wo, ss,
                           n, ident) for c in range(3)]

    wq = p['inter_Wq'] * scale
    wkg = wkg_of(p['inter_Wk'], p['inter_Wg'])
    wv2 = jnp.concatenate([p['inter_Wv'], zv], axis=0)
    gi2 = _sc_gather(tabs[1], topk_i2.astype(jnp.int32).reshape(-1))
    gi3 = _sc_gather(tabs[2], topk_i3.astype(jnp.int32).reshape(-1))
    g1 = _attention(tabs[0], nuvs[0], gi2, wq, wkg, wv2, p['inter_Wo'], ss,
                    n, ident)
    g2 = _attention(tabs[0], nuvs[0], gi3, wq, wkg, wv2, p['inter_Wo'], ss,
                    n, ident)

    return _pool_head(g1, g2, batch_p1.astype(jnp.int32).reshape(n, 1),
                      p['out_w1'].T, p['out_w2'].T, p['out_w3'].T)


# speedup vs baseline: 11.2945x; 1.0146x over previous
import functools

import jax
import jax.numpy as jnp
from jax import lax
from jax.experimental import pallas as pl
from jax.experimental.pallas import tpu as pltpu
from jax.experimental.pallas import tpu_sc as plsc

K = 16
E = 64
H = 4
DH = E // H
DPAD = 128

_HI = jax.lax.Precision.HIGHEST


def _elu(x):
    return jnp.where(x > 0, x, jnp.exp(x) - 1.0)


def _embed_body(tok_ref, llm_ref, xyz_ref, emb_ref, g_ref, b_ref, w1_ref,
                b1_ref, w2_ref, b2_ref, out_ref):
    x = llm_ref[...]
    d = x.shape[1]
    m = jnp.sum(x, axis=1, keepdims=True) * (1.0 / d)
    v = jnp.sum(x * x, axis=1, keepdims=True) * (1.0 / d) - m * m
    h = (x - m) * (lax.rsqrt(v + 1e-5) * g_ref[...]) + b_ref[...]
    h = _elu(jnp.dot(h, w1_ref[...], preferred_element_type=jnp.float32)
             + b1_ref[...])
    h = _elu(jnp.dot(h, w2_ref[...], preferred_element_type=jnp.float32)
             + b2_ref[...])
    tok = tok_ref[...]
    blk = tok.shape[0]
    oh = (tok == lax.broadcasted_iota(jnp.int32, (blk, 32), 1)
          ).astype(jnp.float32)
    ft = jnp.dot(oh, emb_ref[...], precision=_HI,
                 preferred_element_type=jnp.float32)
    xyz = xyz_ref[...]
    pad = jnp.zeros((blk, DPAD - E - 9), jnp.float32)
    out_ref[...] = jnp.concatenate([ft, h, xyz, xyz, xyz, pad], axis=1)


def _embed(tok2d, llm, xyz, emb_pad, ln_g, ln_b, w1, b1, w2, b2):
    n, d = llm.shape
    blk = 1000
    assert n % blk == 0
    return pl.pallas_call(
        _embed_body,
        grid=(n // blk,),
        in_specs=[
            pl.BlockSpec((blk, 1), lambda i: (i, 0)),
            pl.BlockSpec((blk, d), lambda i: (i, 0)),
            pl.BlockSpec((blk, 3), lambda i: (i, 0)),
            pl.BlockSpec((32, 32), lambda i: (0, 0)),
            pl.BlockSpec((1, d), lambda i: (0, 0)),
            pl.BlockSpec((1, d), lambda i: (0, 0)),
            pl.BlockSpec((d, E), lambda i: (0, 0)),
            pl.BlockSpec((1, E), lambda i: (0, 0)),
            pl.BlockSpec((E, 32), lambda i: (0, 0)),
            pl.BlockSpec((1, 32), lambda i: (0, 0)),
        ],
        out_specs=pl.BlockSpec((blk, DPAD), lambda i: (i, 0)),
        out_shape=jax.ShapeDtypeStruct((n, DPAD), jnp.float32),
    )(tok2d, llm, xyz, emb_pad, ln_g, ln_b, w1, b1, w2, b2)


def _sc_gather(table, idx):
    bidx = idx.shape[0]
    info = plsc.get_sparse_core_info()
    nw = info.num_cores * info.num_subcores
    per_w = bidx // nw
    assert per_w * nw == bidx
    ch = 440
    while per_w % ch or ch % 8:
        ch -= 8
    nchunk = per_w // ch
    idx3 = idx.reshape(nw, nchunk, ch)
    mesh = plsc.VectorSubcoreMesh(core_axis_name="c", subcore_axis_name="s")

    @functools.partial(
        pl.kernel, mesh=mesh,
        out_type=jax.ShapeDtypeStruct((nw, nchunk, ch, DPAD), jnp.float32),
        scratch_types=[
            pltpu.VMEM((ch,), jnp.int32),
            pltpu.VMEM((ch,), jnp.int32),
            pltpu.VMEM((ch, DPAD), jnp.float32),
            pltpu.VMEM((ch, DPAD), jnp.float32),
            pltpu.SemaphoreType.DMA,
            pltpu.SemaphoreType.DMA,
        ],
    )
    def k(table_hbm, idx_hbm, out_hbm, idx_a, idx_b, buf_a, buf_b, sem_a,
          sem_b):
        wid = lax.axis_index("s") * info.num_cores + lax.axis_index("c")

        pltpu.sync_copy(idx_hbm.at[wid, 0], idx_a)
        pltpu.async_copy(table_hbm.at[idx_a], buf_a, sem_a)

        def pair(h, carry):
            c0 = 2 * h
            pltpu.sync_copy(idx_hbm.at[wid, c0 + 1], idx_b)
            pltpu.async_copy(table_hbm.at[idx_b], buf_b, sem_b)
            pltpu.make_async_copy(table_hbm.at[idx_a], buf_a, sem_a).wait()
            pltpu.sync_copy(buf_a, out_hbm.at[wid, c0])

            @pl.when(c0 + 2 < nchunk)
            def _():
                pltpu.sync_copy(idx_hbm.at[wid, c0 + 2], idx_a)
                pltpu.async_copy(table_hbm.at[idx_a], buf_a, sem_a)

            pltpu.make_async_copy(table_hbm.at[idx_b], buf_b, sem_b).wait()
            pltpu.sync_copy(buf_b, out_hbm.at[wid, c0 + 1])
            return carry

        lax.fori_loop(0, nchunk // 2, pair, 0)
        if nchunk % 2:
            pltpu.make_async_copy(table_hbm.at[idx_a], buf_a, sem_a).wait()
            pltpu.sync_copy(buf_a, out_hbm.at[wid, nchunk - 1])

    out = k(table, idx3)
    return out.reshape(bidx, DPAD)


def _attn_body(tq_ref, nuv_ref, g_ref, wq_ref, wkgv_ref, wo_ref,
               ss_ref, out_ref):
    tq = tq_ref[...]
    blk = tq.shape[0]
    fq = tq[:, :E]
    G = g_ref[...]
    q = jnp.dot(fq, wq_ref[...], preferred_element_type=jnp.float32)
    zf = jnp.zeros((blk, E), jnp.float32)
    pb = jnp.zeros((blk, DPAD - E - 9), jnp.float32)
    qxa = jnp.concatenate([zf, tq[:, E:]], axis=1)
    qxb = jnp.concatenate([zf + 1.0, nuv_ref[...], pb], axis=1)
    g3 = G.reshape(blk, K, DPAD)
    x = ((g3 - qxa[:, None, :]) * qxb[:, None, :]).reshape(blk * K, DPAD)
    y = jnp.dot(x, wkgv_ref[...], preferred_element_type=jnp.float32)
    nk = y[:, :E]
    nv = y[:, E:]
    p = (nk.reshape(blk, K, E) * q[:, None, :]).reshape(blk * K, E)
    sr = jnp.dot(p, ss_ref[...], preferred_element_type=jnp.float32)
    e3 = jnp.exp(sr).reshape(blk, K, E)
    r = 1.0 / jnp.sum(e3, axis=1, keepdims=True)
    w3 = (e3 * r) * nv.reshape(blk, K, E)
    o = jnp.sum(w3, axis=1)
    nf = fq + jnp.dot(o, wo_ref[...], preferred_element_type=jnp.float32)
    out_ref[...] = jnp.concatenate([nf, tq[:, E:]], axis=1)


def _attention(table, nuv9, g2, wq, wkgv, wo, ss, m, tq_map,
               blk=400):
    nblk = m // blk
    assert nblk * blk == m
    return pl.pallas_call(
        _attn_body,
        grid=(nblk,),
        in_specs=[
            pl.BlockSpec((blk, DPAD), tq_map),
            pl.BlockSpec((blk, 9), tq_map),
            pl.BlockSpec((blk * K, DPAD), lambda i: (i, 0)),
            pl.BlockSpec((E, E), lambda i: (0, 0)),
            pl.BlockSpec((DPAD, 2 * E), lambda i: (0, 0)),
            pl.BlockSpec((E, E), lambda i: (0, 0)),
            pl.BlockSpec((E, E), lambda i: (0, 0)),
        ],
        out_specs=pl.BlockSpec((blk, DPAD), lambda i: (i, 0)),
        out_shape=jax.ShapeDtypeStruct((m, DPAD), jnp.float32),
    )(table, nuv9, g2, wq, wkgv, wo, ss)


def _pool_body(g1_ref, g2_ref, b_ref, w1t_ref, w2t_ref, w3t_ref, out_ref,
               acc_ref):
    i = pl.program_id(0)
    ng = pl.num_programs(0)

    @pl.when(i == 0)
    def _():
        acc_ref[...] = jnp.zeros_like(acc_ref)

    diff = g1_ref[:, :E] - g2_ref[:, :E]
    b = b_ref[...]
    blk = b.shape[0]
    oh = (b == lax.broadcasted_iota(jnp.int32, (blk, 16), 1)
          ).astype(jnp.float32)
    acc_ref[...] += lax.dot_general(diff, oh, (((0,), (0,)), ((), ())),
                                    precision=_HI,
                                    preferred_element_type=jnp.float32)

    @pl.when(i == ng - 1)
    def _():
        t = jnp.dot(w1t_ref[...], acc_ref[...],
                    preferred_element_type=jnp.float32)
        t = jnp.dot(w2t_ref[...], t, preferred_element_type=jnp.float32)
        out_ref[...] = jnp.dot(w3t_ref[...], t,
                               preferred_element_type=jnp.float32)


def _pool_head(g1, g2, batch2d, w1t, w2t, w3t):
    n = batch2d.shape[0]
    blk = 1000
    assert n % blk == 0
    nblk = n // blk
    out = pl.pallas_call(
        _pool_body,
        grid=(nblk,),
        in_specs=[
            pl.BlockSpec((blk, DPAD), lambda i: (i, 0)),
            pl.BlockSpec((blk, DPAD), lambda i: (i, 0)),
            pl.BlockSpec((blk, 1), lambda i: (i, 0)),
            pl.BlockSpec((E, E), lambda i: (0, 0)),
            pl.BlockSpec((E, E), lambda i: (0, 0)),
            pl.BlockSpec((1, E), lambda i: (0, 0)),
        ],
        out_specs=pl.BlockSpec((1, 16), lambda i: (0, 0)),
        out_shape=jax.ShapeDtypeStruct((1, 16), jnp.float32),
        scratch_shapes=[pltpu.VMEM((E, 16), jnp.float32)],
    )(g1, g2, batch2d, w1t, w2t, w3t)
    return out.reshape(16)


def kernel(token_p1, token_p2, token_p3, llm_p1, llm_p2, llm_p3, xyz_p1,
           xyz_p2, xyz_p3, nuv_p1, nuv_p2, nuv_p3, topk_p1, topk_p2, topk_p3,
           topk_i2, topk_i3, batch_p1, params):
    p = params
    n = llm_p1.shape[0]
    emb_pad = jnp.pad(p['emb_tok'].astype(jnp.float32), ((0, 11), (0, 0)))
    ln_g = p['ln_g'].reshape(1, -1)
    ln_b = p['ln_b'].reshape(1, -1)
    b1 = p['llm_b1'].reshape(1, -1)
    b2 = p['llm_b2'].reshape(1, -1)
    ss = jnp.kron(jnp.eye(H, dtype=jnp.float32),
                  jnp.ones((DH, DH), jnp.float32))
    scale = 1.0 / (DH ** 0.5)

    tabs = []
    for tok, llm, xyz in ((token_p1, llm_p1, xyz_p1),
                          (token_p2, llm_p2, xyz_p2),
                          (token_p3, llm_p3, xyz_p3)):
        tabs.append(_embed(tok.astype(jnp.int32).reshape(n, 1), llm, xyz,
                           emb_pad, ln_g, ln_b, p['llm_w1'], b1,
                           p['llm_w2'], b2))

    nuvs = [nuv_p1.reshape(n, 9), nuv_p2.reshape(n, 9), nuv_p3.reshape(n, 9)]
    idxs = [topk_p1.astype(jnp.int32).reshape(-1),
            topk_p2.astype(jnp.int32).reshape(-1),
            topk_p3.astype(jnp.int32).reshape(-1)]

    zkg = jnp.zeros((DPAD - E - 9, E), jnp.float32)
    zv = jnp.zeros((DPAD - E, E), jnp.float32)

    def wkgv_of(wk, wg, wv):
        wkg = jnp.concatenate([wk, jnp.repeat(wg, 3, axis=0), zkg], axis=0)
        wv2 = jnp.concatenate([wv, zv], axis=0)
        return jnp.concatenate([wkg, wv2], axis=1)

    ident = lambda i: (i, 0)
    for l in range(p['stru_Wq'].shape[0]):
        wq = p['stru_Wq'][l] * scale
        wkgv = wkgv_of(p['stru_Wk'][l], p['stru_Wg'][l], p['stru_Wv'][l])
        wo = p['stru_Wo'][l]
        gs = [_sc_gather(tabs[c], idxs[c]) for c in range(3)]
        tabs = [_attention(tabs[c], nuvs[c], gs[c], wq, wkgv, wo, ss,
                           n, ident) for c in range(3)]

    wq = p['inter_Wq'] * scale
    wkgv = wkgv_of(p['inter_Wk'], p['inter_Wg'], p['inter_Wv'])
    gi2 = _sc_gather(tabs[1], topk_i2.astype(jnp.int32).reshape(-1))
    gi3 = _sc_gather(tabs[2], topk_i3.astype(jnp.int32).reshape(-1))
    g1 = _attention(tabs[0], nuvs[0], gi2, wq, wkgv,
                    p['inter_Wo'], ss, n, ident)
    g2 = _attention(tabs[0], nuvs[0], gi3, wq, wkgv,
                    p['inter_Wo'], ss, n, ident)

    return _pool_head(g1, g2, batch_p1.astype(jnp.int32).reshape(n, 1),
                      p['out_w1'].T, p['out_w2'].T, p['out_w3'].T)
